# Initial kernel scaffold; baseline (speedup 1.0000x reference)
import jax
import jax.numpy as jnp
from jax.experimental import pallas as pl


def kernel(output, target):
    raise NotImplementedError("write your pallas kernel here")


# speedup vs baseline: 1.1658x; 1.1658x over previous
import functools

import jax
import jax.numpy as jnp
from jax import lax
from jax.experimental import pallas as pl
from jax.experimental.pallas import tpu as pltpu
from jax.experimental.pallas import tpu_sc as plsc

_VW = 16
_DW = 128
_BC = 2048


def _sc_gather_rows(table, target, batch, n):
  info = plsc.get_sparse_core_info()
  nw = info.num_cores * info.num_subcores
  bpw = batch // nw

  @functools.partial(
      pl.kernel,
      out_type=jax.ShapeDtypeStruct((batch, _DW), jnp.float32),
      mesh=plsc.VectorSubcoreMesh(core_axis_name="c", subcore_axis_name="s"),
      scratch_types=[
          pltpu.VMEM((bpw,), jnp.int32),
          pltpu.VMEM((bpw,), jnp.int32),
          pltpu.VMEM((bpw, _DW), jnp.float32),
          pltpu.SemaphoreType.DMA,
      ],
  )
  def gather_k(table_hbm, tgt_hbm, rows_hbm, tgt_v, idx_v, rows_v, sem):
    wid = lax.axis_index("s") * info.num_cores + lax.axis_index("c")
    base = wid * bpw
    pltpu.sync_copy(tgt_hbm.at[pl.ds(base, bpw)], tgt_v)
    for j in range(bpw // _VW):
      t = tgt_v[pl.ds(j * _VW, _VW)]
      flat = (base + j * _VW + lax.iota(jnp.int32, _VW)) * n + t
      idx_v[pl.ds(j * _VW, _VW)] = lax.shift_right_logical(flat, 7)
    pltpu.async_copy(table_hbm.at[idx_v], rows_v, sem).wait()
    pltpu.sync_copy(rows_v, rows_hbm.at[pl.ds(base, bpw)])

  return gather_k(table, target)


def _tc_count(output, rows, tgt2d, topk):
  batch, n = output.shape
  nb = pl.cdiv(n, _BC)
  inv_b = 1.0 / batch
  nmod = n % _DW

  def body(tgt_ref, rows_ref, x_ref, out_ref, acc_ref, tv_ref):
    p = pl.program_id(0)

    @pl.when(p == 0)
    def _init():
      row_id = lax.broadcasted_iota(jnp.int32, (batch, 1), 0)
      tmod = (row_id * nmod + tgt_ref[...]) & (_DW - 1)
      lane = lax.broadcasted_iota(jnp.int32, (batch, _DW), 1)
      sel = jnp.where(lane == tmod, rows_ref[...], 0.0)
      tv_ref[...] = jnp.sum(sel, axis=1, keepdims=True)
      acc_ref[...] = jnp.zeros_like(acc_ref)

    x = x_ref[...]
    tv = tv_ref[...]
    tgt = tgt_ref[...]
    col = p * _BC + lax.broadcasted_iota(jnp.int32, (batch, _BC), 1)
    ranked_above = (x > tv) | ((x == tv) & (col < tgt))
    hit = jnp.where(ranked_above & (col < n), 1.0, 0.0)
    acc_ref[...] += jnp.sum(hit, axis=1, keepdims=True)

    @pl.when(p == nb - 1)
    def _fin():
      rank = acc_ref[...]
      parts = [
          (jnp.sum(jnp.where(rank < float(k), 1.0, 0.0)) * inv_b).reshape(1, 1)
          for k in topk
      ]
      out_ref[...] = jnp.concatenate(parts, axis=1)

  return pl.pallas_call(
      body,
      grid=(nb,),
      in_specs=[
          pl.BlockSpec((batch, 1), lambda p: (0, 0)),
          pl.BlockSpec((batch, _DW), lambda p: (0, 0)),
          pl.BlockSpec((batch, _BC), lambda p: (0, p)),
      ],
      out_specs=pl.BlockSpec((1, len(topk)), lambda p: (0, 0)),
      out_shape=jax.ShapeDtypeStruct((1, len(topk)), jnp.float32),
      scratch_shapes=[
          pltpu.VMEM((batch, 1), jnp.float32),
          pltpu.VMEM((batch, 1), jnp.float32),
      ],
  )(tgt2d, rows, output)


def kernel(output, target):
  batch, n = output.shape
  tgt = target.astype(jnp.int32)
  table = output.reshape(batch * n // _DW, _DW)
  rows = _sc_gather_rows(table, tgt, batch, n)
  res = _tc_count(output, rows, tgt.reshape(batch, 1), (1, 5))
  return res.reshape(2)


# speedup vs baseline: 2.3684x; 2.0316x over previous
import functools

import jax
import jax.numpy as jnp
from jax import lax
from jax.experimental import pallas as pl
from jax.experimental.pallas import tpu as pltpu
from jax.experimental.pallas import tpu_sc as plsc

_L = 16
_BC = 2048


def _sc_gather_tv(output, target, batch):
  info = plsc.get_sparse_core_info()
  nw = info.num_cores * info.num_subcores
  bpw = batch // nw

  @functools.partial(
      pl.kernel,
      out_type=jax.ShapeDtypeStruct((batch,), jnp.float32),
      mesh=plsc.VectorSubcoreMesh(core_axis_name="c", subcore_axis_name="s"),
      compiler_params=pltpu.CompilerParams(needs_layout_passes=False),
      scratch_types=[
          pltpu.VMEM((bpw,), jnp.int32),
          pltpu.VMEM((bpw * 8, 128), jnp.float32),
          pltpu.VMEM((bpw,), jnp.float32),
          pltpu.SemaphoreType.DMA,
      ],
  )
  def gather_k(out_hbm, tgt_hbm, tv_hbm, tgt_v, tiles_v, tv_v, sem):
    wid = lax.axis_index("s") * info.num_cores + lax.axis_index("c")
    base = wid * bpw
    pltpu.sync_copy(tgt_hbm.at[pl.ds(base, bpw)], tgt_v)
    t_groups = [tgt_v[pl.ds(g * _L, _L)] for g in range(bpw // _L)]
    copies = []
    for j in range(bpw):
      t = t_groups[j // _L][j % _L]
      col = pl.multiple_of((t >> 7) << 7, 128)
      row = pl.multiple_of(base + 8 * (j // 8), 8)
      copies.append(
          pltpu.make_async_copy(
              out_hbm.at[pl.ds(row, 8), pl.ds(col, 128)],
              tiles_v.at[pl.ds(j * 8, 8)],
              sem,
          )
      )
    for c in copies:
      c.start()
    for c in copies:
      c.wait()
    for g in range(bpw // _L):
      j_vec = g * _L + lax.iota(jnp.int32, _L)
      t_vec = tgt_v[pl.ds(g * _L, _L)]
      row_idx = j_vec * 8 + (j_vec & 7)
      col_idx = t_vec & 127
      tv_v[pl.ds(g * _L, _L)] = plsc.load_gather(tiles_v, [row_idx, col_idx])
    pltpu.sync_copy(tv_v, tv_hbm.at[pl.ds(base, bpw)])

  return gather_k(output, target)


def _tc_count(output, tv2d, tgt2d, topk):
  batch, n = output.shape
  nb = pl.cdiv(n, _BC)
  inv_b = 1.0 / batch

  def body(tgt_ref, tv_ref, x_ref, out_ref, acc_ref):
    p = pl.program_id(0)

    @pl.when(p == 0)
    def _init():
      acc_ref[...] = jnp.zeros_like(acc_ref)

    x = x_ref[...]
    tv = tv_ref[...]
    iota = lax.broadcasted_iota(jnp.int32, (batch, _BC), 1)
    tgt_local = tgt_ref[...] - p * _BC
    above = (x > tv) | ((x == tv) & (iota < tgt_local))
    hit = above & (iota < (n - p * _BC))
    acc_ref[...] += jnp.sum(jnp.where(hit, 1.0, 0.0), axis=1, keepdims=True)

    @pl.when(p == nb - 1)
    def _fin():
      rank = acc_ref[...]
      parts = [
          (jnp.sum(jnp.where(rank < float(k), 1.0, 0.0)) * inv_b).reshape(1, 1)
          for k in topk
      ]
      out_ref[...] = jnp.concatenate(parts, axis=1)

  return pl.pallas_call(
      body,
      grid=(nb,),
      in_specs=[
          pl.BlockSpec((batch, 1), lambda p: (0, 0)),
          pl.BlockSpec((batch, 1), lambda p: (0, 0)),
          pl.BlockSpec((batch, _BC), lambda p: (0, p)),
      ],
      out_specs=pl.BlockSpec((1, len(topk)), lambda p: (0, 0)),
      out_shape=jax.ShapeDtypeStruct((1, len(topk)), jnp.float32),
      scratch_shapes=[
          pltpu.VMEM((batch, 1), jnp.float32),
      ],
  )(tgt2d, tv2d, output)


def kernel(output, target):
  batch, n = output.shape
  tgt = target.astype(jnp.int32)
  tv = _sc_gather_tv(output, tgt, batch)
  res = _tc_count(output, tv.reshape(batch, 1), tgt.reshape(batch, 1), (1, 5))
  return res.reshape(2)


# speedup vs baseline: 2.3981x; 1.0125x over previous
import functools

import jax
import jax.numpy as jnp
from jax import lax
from jax.experimental import pallas as pl
from jax.experimental.pallas import tpu as pltpu
from jax.experimental.pallas import tpu_sc as plsc

_L = 16
_BC = 4096


def _sc_gather_tv(output, target, batch):
  info = plsc.get_sparse_core_info()
  nw = info.num_cores * info.num_subcores
  bpw = batch // nw

  @functools.partial(
      pl.kernel,
      out_type=jax.ShapeDtypeStruct((batch,), jnp.float32),
      mesh=plsc.VectorSubcoreMesh(core_axis_name="c", subcore_axis_name="s"),
      compiler_params=pltpu.CompilerParams(needs_layout_passes=False),
      scratch_types=[
          pltpu.VMEM((bpw,), jnp.int32),
          pltpu.VMEM((bpw * 8, 128), jnp.float32),
          pltpu.VMEM((bpw,), jnp.float32),
          pltpu.SemaphoreType.DMA,
      ],
  )
  def gather_k(out_hbm, tgt_hbm, tv_hbm, tgt_v, tiles_v, tv_v, sem):
    wid = lax.axis_index("s") * info.num_cores + lax.axis_index("c")
    base = wid * bpw
    pltpu.sync_copy(tgt_hbm.at[pl.ds(base, bpw)], tgt_v)
    t_groups = [tgt_v[pl.ds(g * _L, _L)] for g in range(bpw // _L)]
    copies = []
    for j in range(bpw):
      t = t_groups[j // _L][j % _L]
      col = pl.multiple_of((t >> 7) << 7, 128)
      row = pl.multiple_of(base + 8 * (j // 8), 8)
      copies.append(
          pltpu.make_async_copy(
              out_hbm.at[pl.ds(row, 8), pl.ds(col, 128)],
              tiles_v.at[pl.ds(j * 8, 8)],
              sem,
          )
      )
    for c in copies:
      c.start()
    for c in copies:
      c.wait()
    for g in range(bpw // _L):
      j_vec = g * _L + lax.iota(jnp.int32, _L)
      t_vec = tgt_v[pl.ds(g * _L, _L)]
      row_idx = j_vec * 8 + (j_vec & 7)
      col_idx = t_vec & 127
      tv_v[pl.ds(g * _L, _L)] = plsc.load_gather(tiles_v, [row_idx, col_idx])
    pltpu.sync_copy(tv_v, tv_hbm.at[pl.ds(base, bpw)])

  return gather_k(output, target)


def _tc_count(output, tv2d, tgt2d, topk):
  batch, n = output.shape
  nb = pl.cdiv(n, _BC)
  half = batch // 2
  inv_b = 1.0 / batch

  def body(tgt_ref, tv_ref, x_ref, rank_ref):
    p = pl.program_id(1)

    @pl.when(p == 0)
    def _init():
      rank_ref[...] = jnp.zeros_like(rank_ref)

    x = x_ref[...]
    tv = tv_ref[...]
    iota = lax.broadcasted_iota(jnp.int32, (half, _BC), 1)
    tgt_local = tgt_ref[...] - p * _BC
    above = (x > tv) | ((x == tv) & (iota < tgt_local))

    def accum(mask):
      c = jnp.where(mask, 1.0, 0.0)
      s = c[:, 0:128]
      for i in range(1, _BC // 128):
        s = s + c[:, i * 128:(i + 1) * 128]
      rank_ref[...] += s

    @pl.when(p < nb - 1)
    def _full():
      accum(above)

    @pl.when(p == nb - 1)
    def _last():
      accum(above & (iota < (n - p * _BC)))

  rank = pl.pallas_call(
      body,
      grid=(2, nb),
      in_specs=[
          pl.BlockSpec((half, 1), lambda h, p: (h, 0)),
          pl.BlockSpec((half, 1), lambda h, p: (h, 0)),
          pl.BlockSpec((half, _BC), lambda h, p: (h, p)),
      ],
      out_specs=pl.BlockSpec((half, 128), lambda h, p: (h, 0)),
      out_shape=jax.ShapeDtypeStruct((batch, 128), jnp.float32),
      compiler_params=pltpu.CompilerParams(
          dimension_semantics=("parallel", "arbitrary")
      ),
  )(tgt2d, tv2d, output)

  def fin_body(rank_ref, out_ref):
    r = jnp.sum(rank_ref[...], axis=1, keepdims=True)
    parts = [
        (jnp.sum(jnp.where(r < float(k), 1.0, 0.0)) * inv_b).reshape(1, 1)
        for k in topk
    ]
    out_ref[...] = jnp.concatenate(parts, axis=1)

  return pl.pallas_call(
      fin_body,
      out_shape=jax.ShapeDtypeStruct((1, len(topk)), jnp.float32),
  )(rank)


def kernel(output, target):
  batch, n = output.shape
  tgt = target.astype(jnp.int32)
  tv = _sc_gather_tv(output, tgt, batch)
  res = _tc_count(output, tv.reshape(batch, 1), tgt.reshape(batch, 1), (1, 5))
  return res.reshape(2)


# speedup vs baseline: 2.4930x; 1.0396x over previous
import functools

import jax
import jax.numpy as jnp
from jax import lax
from jax.experimental import pallas as pl
from jax.experimental.pallas import tpu as pltpu
from jax.experimental.pallas import tpu_sc as plsc

_L = 16
_BC = 4096


def _sc_gather_tv(output, target, batch):
  info = plsc.get_sparse_core_info()
  nw = info.num_cores * info.num_subcores
  bpw = batch // nw

  @functools.partial(
      pl.kernel,
      out_type=jax.ShapeDtypeStruct((batch,), jnp.float32),
      mesh=plsc.VectorSubcoreMesh(core_axis_name="c", subcore_axis_name="s"),
      compiler_params=pltpu.CompilerParams(needs_layout_passes=False),
      scratch_types=[
          pltpu.VMEM((bpw,), jnp.int32),
          pltpu.VMEM((bpw * 8, 128), jnp.float32),
          pltpu.VMEM((bpw,), jnp.float32),
          pltpu.SemaphoreType.DMA,
      ],
  )
  def gather_k(out_hbm, tgt_hbm, tv_hbm, tgt_v, tiles_v, tv_v, sem):
    wid = lax.axis_index("s") * info.num_cores + lax.axis_index("c")
    base = wid * bpw
    pltpu.sync_copy(tgt_hbm.at[pl.ds(base, bpw)], tgt_v)
    t_groups = [tgt_v[pl.ds(g * _L, _L)] for g in range(bpw // _L)]
    copies = []
    for j in range(bpw):
      t = t_groups[j // _L][j % _L]
      col = pl.multiple_of((t >> 7) << 7, 128)
      row = pl.multiple_of(base + 8 * (j // 8), 8)
      copies.append(
          pltpu.make_async_copy(
              out_hbm.at[pl.ds(row, 8), pl.ds(col, 128)],
              tiles_v.at[pl.ds(j * 8, 8)],
              sem,
          )
      )
    for c in copies:
      c.start()
    for c in copies:
      c.wait()
    for g in range(bpw // _L):
      j_vec = g * _L + lax.iota(jnp.int32, _L)
      t_vec = tgt_v[pl.ds(g * _L, _L)]
      row_idx = j_vec * 8 + (j_vec & 7)
      col_idx = t_vec & 127
      tv_v[pl.ds(g * _L, _L)] = plsc.load_gather(tiles_v, [row_idx, col_idx])
    pltpu.sync_copy(tv_v, tv_hbm.at[pl.ds(base, bpw)])

  return gather_k(output, target)


def _tc_count(output, tvlo2d, tvhi2d, tgt2d, topk):
  batch, n = output.shape
  nb = pl.cdiv(n, _BC)
  half = batch // 2
  inv_b = 1.0 / batch

  def body(tgt_ref, tvlo_ref, tvhi_ref, x_ref, rank_ref):
    p = pl.program_id(1)

    @pl.when(p == 0)
    def _init():
      rank_ref[...] = jnp.zeros_like(rank_ref)

    x = x_ref[...]
    iota = lax.broadcasted_iota(jnp.int32, (half, _BC), 1)
    tgt_local = tgt_ref[...] - p * _BC
    thr = jnp.where(iota < tgt_local, tvlo_ref[...], tvhi_ref[...])

    def accum(thr):
      c = jnp.where(x >= thr, 1.0, 0.0)
      s = c[:, 0:128]
      for i in range(1, _BC // 128):
        s = s + c[:, i * 128:(i + 1) * 128]
      rank_ref[...] += s

    @pl.when(p < nb - 1)
    def _full():
      accum(thr)

    @pl.when(p == nb - 1)
    def _last():
      accum(jnp.where(iota < (n - p * _BC), thr, jnp.inf))

  rank = pl.pallas_call(
      body,
      grid=(2, nb),
      in_specs=[
          pl.BlockSpec((half, 1), lambda h, p: (h, 0)),
          pl.BlockSpec((half, 1), lambda h, p: (h, 0)),
          pl.BlockSpec((half, 1), lambda h, p: (h, 0)),
          pl.BlockSpec((half, _BC), lambda h, p: (h, p)),
      ],
      out_specs=pl.BlockSpec((half, 128), lambda h, p: (h, 0)),
      out_shape=jax.ShapeDtypeStruct((batch, 128), jnp.float32),
      compiler_params=pltpu.CompilerParams(
          dimension_semantics=("parallel", "arbitrary")
      ),
  )(tgt2d, tvlo2d, tvhi2d, output)

  def fin_body(rank_ref, out_ref):
    r = jnp.sum(rank_ref[...], axis=1, keepdims=True)
    parts = [
        (jnp.sum(jnp.where(r < float(k), 1.0, 0.0)) * inv_b).reshape(1, 1)
        for k in topk
    ]
    out_ref[...] = jnp.concatenate(parts, axis=1)

  return pl.pallas_call(
      fin_body,
      out_shape=jax.ShapeDtypeStruct((1, len(topk)), jnp.float32),
  )(rank)


def kernel(output, target):
  batch, n = output.shape
  tgt = target.astype(jnp.int32)
  tv = _sc_gather_tv(output, tgt, batch)
  u = lax.bitcast_convert_type(tv, jnp.int32)
  hi = jnp.where(u >= 0, u + 1, u - 1)
  hi = jnp.where(u == jnp.int32(-2147483648), jnp.int32(1), hi)
  tv_hi = lax.bitcast_convert_type(hi, jnp.float32)
  res = _tc_count(
      output,
      tv.reshape(batch, 1),
      tv_hi.reshape(batch, 1),
      tgt.reshape(batch, 1),
      (1, 5),
  )
  return res.reshape(2)
